# Initial kernel scaffold; baseline (speedup 1.0000x reference)
import jax
import jax.numpy as jnp
from jax.experimental import pallas as pl


def kernel(src, batch, dim_size):
    raise NotImplementedError("write your pallas kernel here")


# speedup vs baseline: 4.4598x; 4.4598x over previous
import functools

import jax
import jax.numpy as jnp
from jax import lax
from jax.experimental import pallas as pl
from jax.experimental.pallas import tpu as pltpu
from jax.experimental.pallas import tpu_sc as plsc

N_ROWS = 320000
N_SEG = 10000
D = 128
W = 128
N_WIN = N_ROWS // W
N_WORKERS = 32
CHUNK = 624
TAIL = N_SEG - 16 * CHUNK
STEPS = (N_WIN + N_WORKERS - 1) // N_WORKERS


def _sc_body(src_hbm, batch_hbm, zeros_hbm, out_hbm, rbuf, ids, acc):
    c = lax.axis_index("c")
    s = lax.axis_index("s")
    wid = s * 2 + c

    pltpu.sync_copy(zeros_hbm.at[pl.ds(s * CHUNK, CHUNK)],
                    acc.at[pl.ds(s * CHUNK, CHUNK)])

    @pl.when(s == 15)
    def _():
        pltpu.sync_copy(zeros_hbm.at[pl.ds(16 * CHUNK, TAIL)],
                        acc.at[pl.ds(16 * CHUNK, TAIL)])

    plsc.subcore_barrier()

    def step(k, carry):
        win = wid + k * N_WORKERS

        @pl.when(win < N_WIN)
        def _():
            pltpu.sync_copy(batch_hbm.at[pl.ds(win * W, W)], ids)
            pltpu.sync_copy(src_hbm.at[pl.ds(win * W, W)], rbuf)
            pltpu.sync_copy(rbuf, acc.at[ids], add=True)

        return carry

    lax.fori_loop(0, STEPS, step, 0)

    plsc.subcore_barrier()
    pltpu.sync_copy(acc.at[pl.ds(s * CHUNK, CHUNK)],
                    out_hbm.at[pl.ds(c * N_SEG + s * CHUNK, CHUNK)])

    @pl.when(s == 15)
    def _():
        pltpu.sync_copy(acc.at[pl.ds(16 * CHUNK, TAIL)],
                        out_hbm.at[pl.ds(c * N_SEG + 16 * CHUNK, TAIL)])


@functools.partial(
    pl.kernel,
    out_type=jax.ShapeDtypeStruct((2 * N_SEG, D), jnp.float32),
    mesh=plsc.VectorSubcoreMesh(core_axis_name="c", subcore_axis_name="s"),
    scratch_types=[
        pltpu.VMEM((W, D), jnp.float32),
        pltpu.VMEM((W,), jnp.int32),
        pltpu.VMEM_SHARED((N_SEG, D), jnp.float32),
    ],
)
def _sc_scatter_add(src_hbm, batch_hbm, zeros_hbm, out_hbm, rbuf, ids, acc):
    _sc_body(src_hbm, batch_hbm, zeros_hbm, out_hbm, rbuf, ids, acc)


def _add_body(a_ref, b_ref, o_ref):
    o_ref[...] = a_ref[...] + b_ref[...]


def _combine(partials):
    blk = 1000
    return pl.pallas_call(
        _add_body,
        grid=(N_SEG // blk,),
        in_specs=[
            pl.BlockSpec((blk, D), lambda i: (i, 0)),
            pl.BlockSpec((blk, D), lambda i: (N_SEG // blk + i, 0)),
        ],
        out_specs=pl.BlockSpec((blk, D), lambda i: (i, 0)),
        out_shape=jax.ShapeDtypeStruct((N_SEG, D), jnp.float32),
    )(partials, partials)


def kernel(src, batch, dim_size):
    batch32 = jnp.asarray(batch, jnp.int32)
    zeros = jnp.zeros((N_SEG, D), jnp.float32)
    partials = _sc_scatter_add(src, batch32, zeros)
    return _combine(partials)


# speedup vs baseline: 7.5357x; 1.6897x over previous
import functools

import jax
import jax.numpy as jnp
from jax import lax
from jax.experimental import pallas as pl
from jax.experimental.pallas import tpu as pltpu
from jax.experimental.pallas import tpu_sc as plsc

N_ROWS = 320000
N_SEG = 10000
D = 128
W = 128
N_WIN = N_ROWS // W
N_WORKERS = 32
CHUNK = 624
TAIL = N_SEG - 16 * CHUNK
STEPS = (N_WIN + N_WORKERS - 1) // N_WORKERS
PAIR_STEPS = (STEPS + 1) // 2


def _sc_body(src_hbm, batch_hbm, zeros_hbm, out_hbm,
             rbuf0, rbuf1, ids0, ids1, acc, sem0, sem1):
    c = lax.axis_index("c")
    s = lax.axis_index("s")
    wid = s * 2 + c

    pltpu.sync_copy(zeros_hbm.at[pl.ds(s * CHUNK, CHUNK)],
                    acc.at[pl.ds(s * CHUNK, CHUNK)])

    @pl.when(s == 15)
    def _():
        pltpu.sync_copy(zeros_hbm.at[pl.ds(16 * CHUNK, TAIL)],
                        acc.at[pl.ds(16 * CHUNK, TAIL)])

    plsc.subcore_barrier()

    ids = [ids0, ids1]
    rbuf = [rbuf0, rbuf1]
    sem = [sem0, sem1]

    def start(k, b):
        win = wid + k * N_WORKERS

        @pl.when(win < N_WIN)
        def _():
            pltpu.async_copy(batch_hbm.at[pl.ds(win * W, W)], ids[b], sem[b])
            pltpu.async_copy(src_hbm.at[pl.ds(win * W, W)], rbuf[b], sem[b])

    def drain_and_scatter(k, b):
        win = wid + k * N_WORKERS

        @pl.when(win < N_WIN)
        def _():
            pltpu.make_async_copy(batch_hbm.at[pl.ds(0, W)], ids[b],
                                  sem[b]).wait()
            pltpu.make_async_copy(src_hbm.at[pl.ds(0, W)], rbuf[b],
                                  sem[b]).wait()
            pltpu.sync_copy(rbuf[b], acc.at[ids[b]], add=True)

    start(0, 0)

    def step(i, carry):
        k = i * 2
        start(k + 1, 1)
        drain_and_scatter(k, 0)
        start(k + 2, 0)
        drain_and_scatter(k + 1, 1)
        return carry

    lax.fori_loop(0, PAIR_STEPS, step, 0)

    plsc.subcore_barrier()
    pltpu.sync_copy(acc.at[pl.ds(s * CHUNK, CHUNK)],
                    out_hbm.at[pl.ds(c * N_SEG + s * CHUNK, CHUNK)])

    @pl.when(s == 15)
    def _():
        pltpu.sync_copy(acc.at[pl.ds(16 * CHUNK, TAIL)],
                        out_hbm.at[pl.ds(c * N_SEG + 16 * CHUNK, TAIL)])


@functools.partial(
    pl.kernel,
    out_type=jax.ShapeDtypeStruct((2 * N_SEG, D), jnp.float32),
    mesh=plsc.VectorSubcoreMesh(core_axis_name="c", subcore_axis_name="s"),
    scratch_types=[
        pltpu.VMEM((W, D), jnp.float32),
        pltpu.VMEM((W, D), jnp.float32),
        pltpu.VMEM((W,), jnp.int32),
        pltpu.VMEM((W,), jnp.int32),
        pltpu.VMEM_SHARED((N_SEG, D), jnp.float32),
        pltpu.SemaphoreType.DMA,
        pltpu.SemaphoreType.DMA,
    ],
)
def _sc_scatter_add(src_hbm, batch_hbm, zeros_hbm, out_hbm,
                    rbuf0, rbuf1, ids0, ids1, acc, sem0, sem1):
    _sc_body(src_hbm, batch_hbm, zeros_hbm, out_hbm,
             rbuf0, rbuf1, ids0, ids1, acc, sem0, sem1)


def _add_body(a_ref, b_ref, o_ref):
    o_ref[...] = a_ref[...] + b_ref[...]


def _combine(partials):
    blk = 1000
    return pl.pallas_call(
        _add_body,
        grid=(N_SEG // blk,),
        in_specs=[
            pl.BlockSpec((blk, D), lambda i: (i, 0)),
            pl.BlockSpec((blk, D), lambda i: (N_SEG // blk + i, 0)),
        ],
        out_specs=pl.BlockSpec((blk, D), lambda i: (i, 0)),
        out_shape=jax.ShapeDtypeStruct((N_SEG, D), jnp.float32),
    )(partials, partials)


def kernel(src, batch, dim_size):
    batch32 = jnp.asarray(batch, jnp.int32)
    zeros = jnp.zeros((N_SEG, D), jnp.float32)
    partials = _sc_scatter_add(src, batch32, zeros)
    return _combine(partials)
